# Initial kernel scaffold; baseline (speedup 1.0000x reference)
import jax
import jax.numpy as jnp
from jax.experimental import pallas as pl


def kernel(entity_ids, table):
    raise NotImplementedError("write your pallas kernel here")


# speedup vs baseline: 1.2903x; 1.2903x over previous
import functools

import jax
import jax.numpy as jnp
from jax import lax
from jax.experimental import pallas as pl
from jax.experimental.pallas import tpu as pltpu
from jax.experimental.pallas import tpu_sc as plsc

B = 100000
D = 128
NW = 32
CHUNK = 128
PW = 3128
N_FULL = PW // CHUNK
TAIL_OFF = PW - CHUNK
CHUNK_OFFS = tuple(j * CHUNK for j in range(N_FULL)) + (TAIL_OFF,)
LAST_BASE = B - PW


def _sc_gather(ids, table):
    mesh = plsc.VectorSubcoreMesh(core_axis_name="c", subcore_axis_name="s")

    @functools.partial(
        pl.kernel,
        mesh=mesh,
        out_type=jax.ShapeDtypeStruct((B, D), jnp.float32),
        scratch_types=[
            pltpu.VMEM((len(CHUNK_OFFS), CHUNK), jnp.int32),
            pltpu.VMEM((CHUNK, D), jnp.float32),
            pltpu.SemaphoreType.DMA,
        ],
    )
    def k(ids_hbm, table_hbm, out_hbm, idx_v, rows_v, sem):
        wid = lax.axis_index("s") * 2 + lax.axis_index("c")
        base = jnp.where(wid == NW - 1, LAST_BASE, wid * PW)
        for j, off in enumerate(CHUNK_OFFS):
            pltpu.sync_copy(ids_hbm.at[pl.ds(base + off, CHUNK)], idx_v.at[j])
        for j, off in enumerate(CHUNK_OFFS):
            pltpu.async_copy(table_hbm.at[idx_v.at[j]], rows_v, sem).wait()
            pltpu.sync_copy(rows_v, out_hbm.at[pl.ds(base + off, CHUNK)])

    return k(ids, table)


def kernel(entity_ids, table):
    ids = jnp.squeeze(entity_ids).astype(jnp.int32)
    return _sc_gather(ids, table)


# speedup vs baseline: 1.8971x; 1.4703x over previous
import functools

import jax
import jax.numpy as jnp
from jax import lax
from jax.experimental import pallas as pl
from jax.experimental.pallas import tpu as pltpu
from jax.experimental.pallas import tpu_sc as plsc

B = 100000
D = 128
NW = 32
CHUNK = 128
PW = 3128
N_FULL = PW // CHUNK
TAIL_OFF = PW - CHUNK
CHUNK_OFFS = tuple(j * CHUNK for j in range(N_FULL)) + (TAIL_OFF,)
LAST_BASE = B - PW


def _sc_gather(ids, table):
    mesh = plsc.VectorSubcoreMesh(core_axis_name="c", subcore_axis_name="s")

    @functools.partial(
        pl.kernel,
        mesh=mesh,
        out_type=jax.ShapeDtypeStruct((B, D), jnp.float32),
        scratch_types=[
            pltpu.VMEM((len(CHUNK_OFFS), CHUNK), jnp.int32),
            pltpu.VMEM((2, CHUNK, D), jnp.float32),
            pltpu.SemaphoreType.DMA,
            pltpu.SemaphoreType.DMA,
            pltpu.SemaphoreType.DMA,
            pltpu.SemaphoreType.DMA,
            pltpu.SemaphoreType.DMA,
        ],
    )
    def k(ids_hbm, table_hbm, out_hbm, idx_v, rows_v, isem, gsem0, gsem1,
          ssem0, ssem1):
        wid = lax.axis_index("s") * 2 + lax.axis_index("c")
        base = jnp.where(wid == NW - 1, LAST_BASE, wid * PW)
        idx_cps = [
            pltpu.async_copy(ids_hbm.at[pl.ds(base + off, CHUNK)],
                             idx_v.at[j], isem)
            for j, off in enumerate(CHUNK_OFFS)
        ]
        for cp in idx_cps:
            cp.wait()
        gsems = (gsem0, gsem1)
        ssems = (ssem0, ssem1)
        gath = [None, None]
        scat = [None, None]
        for j, off in enumerate(CHUNK_OFFS):
            b = j & 1
            if scat[b] is not None:
                scat[b].wait()
            gath[b] = pltpu.async_copy(table_hbm.at[idx_v.at[j]],
                                       rows_v.at[b], gsems[b])
            if j > 0:
                pb = (j - 1) & 1
                gath[pb].wait()
                poff = CHUNK_OFFS[j - 1]
                scat[pb] = pltpu.async_copy(
                    rows_v.at[pb], out_hbm.at[pl.ds(base + poff, CHUNK)],
                    ssems[pb])
        lb = (len(CHUNK_OFFS) - 1) & 1
        gath[lb].wait()
        scat[lb] = pltpu.async_copy(
            rows_v.at[lb], out_hbm.at[pl.ds(base + CHUNK_OFFS[-1], CHUNK)],
            ssems[lb])
        for s in scat:
            s.wait()

    return k(ids, table)


def kernel(entity_ids, table):
    ids = jnp.squeeze(entity_ids).astype(jnp.int32)
    return _sc_gather(ids, table)


# speedup vs baseline: 1.9720x; 1.0395x over previous
import functools

import jax
import jax.numpy as jnp
from jax import lax
from jax.experimental import pallas as pl
from jax.experimental.pallas import tpu as pltpu
from jax.experimental.pallas import tpu_sc as plsc

B = 100000
D = 128
NW = 32
CHUNK = 128
PW = 3128
N_FULL = PW // CHUNK
TAIL_OFF = PW - CHUNK
CHUNK_OFFS = tuple(j * CHUNK for j in range(N_FULL)) + (TAIL_OFF,)
LAST_BASE = B - PW
NBUF = 4
GDEPTH = 3


def _sc_gather(ids, table):
    mesh = plsc.VectorSubcoreMesh(core_axis_name="c", subcore_axis_name="s")

    @functools.partial(
        pl.kernel,
        mesh=mesh,
        out_type=jax.ShapeDtypeStruct((B, D), jnp.float32),
        scratch_types=(
            [pltpu.VMEM((len(CHUNK_OFFS), CHUNK), jnp.int32),
             pltpu.VMEM((NBUF, CHUNK, D), jnp.float32)]
            + [pltpu.SemaphoreType.DMA] * (1 + 2 * NBUF)
        ),
    )
    def k(ids_hbm, table_hbm, out_hbm, idx_v, rows_v, isem, *sems):
        gsems = sems[:NBUF]
        ssems = sems[NBUF:]
        wid = lax.axis_index("s") * 2 + lax.axis_index("c")
        base = jnp.where(wid == NW - 1, LAST_BASE, wid * PW)
        idx_cps = [
            pltpu.async_copy(ids_hbm.at[pl.ds(base + off, CHUNK)],
                             idx_v.at[j], isem)
            for j, off in enumerate(CHUNK_OFFS)
        ]
        for cp in idx_cps:
            cp.wait()
        n = len(CHUNK_OFFS)
        gath = [None] * n
        scat = [None] * n

        def fire_gather(j):
            b = j % NBUF
            if j >= NBUF:
                scat[j - NBUF].wait()
            gath[j] = pltpu.async_copy(table_hbm.at[idx_v.at[j]],
                                       rows_v.at[b], gsems[b])

        def fire_scatter(j):
            b = j % NBUF
            gath[j].wait()
            scat[j] = pltpu.async_copy(
                rows_v.at[b], out_hbm.at[pl.ds(base + CHUNK_OFFS[j], CHUNK)],
                ssems[b])

        for j in range(GDEPTH):
            fire_gather(j)
        for j in range(n):
            if j + GDEPTH < n:
                fire_gather(j + GDEPTH)
            fire_scatter(j)
        for j in range(n - NBUF, n):
            scat[j].wait()

    return k(ids, table)


def kernel(entity_ids, table):
    ids = jnp.squeeze(entity_ids).astype(jnp.int32)
    return _sc_gather(ids, table)


# speedup vs baseline: 1.9920x; 1.0102x over previous
import functools

import jax
import jax.numpy as jnp
from jax import lax
from jax.experimental import pallas as pl
from jax.experimental.pallas import tpu as pltpu
from jax.experimental.pallas import tpu_sc as plsc

B = 100000
D = 128
NW = 32
CHUNK = 128
PW = 3128
N_FULL = PW // CHUNK
TAIL_OFF = PW - CHUNK
CHUNK_OFFS = tuple(j * CHUNK for j in range(N_FULL)) + (TAIL_OFF,)
LAST_BASE = B - PW
NBUF = 6
GDEPTH = 5


def _sc_gather(ids, table):
    mesh = plsc.VectorSubcoreMesh(core_axis_name="c", subcore_axis_name="s")

    @functools.partial(
        pl.kernel,
        mesh=mesh,
        out_type=jax.ShapeDtypeStruct((B, D), jnp.float32),
        scratch_types=(
            [pltpu.VMEM((len(CHUNK_OFFS), CHUNK), jnp.int32),
             pltpu.VMEM((NBUF, CHUNK, D), jnp.float32)]
            + [pltpu.SemaphoreType.DMA] * (1 + 2 * NBUF)
        ),
    )
    def k(ids_hbm, table_hbm, out_hbm, idx_v, rows_v, isem, *sems):
        gsems = sems[:NBUF]
        ssems = sems[NBUF:]
        wid = lax.axis_index("s") * 2 + lax.axis_index("c")
        base = jnp.where(wid == NW - 1, LAST_BASE, wid * PW)
        idx_cps = [
            pltpu.async_copy(ids_hbm.at[pl.ds(base + off, CHUNK)],
                             idx_v.at[j], isem)
            for j, off in enumerate(CHUNK_OFFS)
        ]
        n = len(CHUNK_OFFS)
        gath = [None] * n
        scat = [None] * n

        def fire_gather(j):
            b = j % NBUF
            idx_cps[j].wait()
            if j >= NBUF:
                scat[j - NBUF].wait()
            gath[j] = pltpu.async_copy(table_hbm.at[idx_v.at[j]],
                                       rows_v.at[b], gsems[b])

        def fire_scatter(j):
            b = j % NBUF
            gath[j].wait()
            scat[j] = pltpu.async_copy(
                rows_v.at[b], out_hbm.at[pl.ds(base + CHUNK_OFFS[j], CHUNK)],
                ssems[b])

        for j in range(GDEPTH):
            fire_gather(j)
        for j in range(n):
            if j + GDEPTH < n:
                fire_gather(j + GDEPTH)
            fire_scatter(j)
        for j in range(n - NBUF, n):
            scat[j].wait()

    return k(ids, table)


def kernel(entity_ids, table):
    ids = jnp.squeeze(entity_ids).astype(jnp.int32)
    return _sc_gather(ids, table)


# speedup vs baseline: 2.0519x; 1.0300x over previous
import functools

import jax
import jax.numpy as jnp
from jax import lax
from jax.experimental import pallas as pl
from jax.experimental.pallas import tpu as pltpu
from jax.experimental.pallas import tpu_sc as plsc

B = 100000
D = 128
NW = 32
CHUNK = 128
PW = 3128
N_FULL = PW // CHUNK
CHUNKS = tuple((j * CHUNK, CHUNK) for j in range(N_FULL)) + (
    (N_FULL * CHUNK, PW - N_FULL * CHUNK),)
LAST_BASE = B - PW
NBUF = 6
GDEPTH = 5


def _sc_gather(ids, table):
    mesh = plsc.VectorSubcoreMesh(core_axis_name="c", subcore_axis_name="s")

    @functools.partial(
        pl.kernel,
        mesh=mesh,
        out_type=jax.ShapeDtypeStruct((B, D), jnp.float32),
        scratch_types=(
            [pltpu.VMEM((PW,), jnp.int32),
             pltpu.VMEM((NBUF, CHUNK, D), jnp.float32)]
            + [pltpu.SemaphoreType.DMA] * (1 + 2 * NBUF)
        ),
    )
    def k(ids_hbm, table_hbm, out_hbm, idx_v, rows_v, isem, *sems):
        gsems = sems[:NBUF]
        ssems = sems[NBUF:]
        wid = lax.axis_index("s") * 2 + lax.axis_index("c")
        base = jnp.where(wid == NW - 1, LAST_BASE, wid * PW)
        pltpu.async_copy(ids_hbm.at[pl.ds(base, PW)], idx_v, isem).wait()
        n = len(CHUNKS)
        gath = [None] * n
        scat = [None] * n

        def fire_gather(j):
            b = j % NBUF
            off, sz = CHUNKS[j]
            if j >= NBUF:
                scat[j - NBUF].wait()
            gath[j] = pltpu.async_copy(
                table_hbm.at[idx_v.at[pl.ds(off, sz)]],
                rows_v.at[b, pl.ds(0, sz)], gsems[b])

        def fire_scatter(j):
            b = j % NBUF
            off, sz = CHUNKS[j]
            gath[j].wait()
            scat[j] = pltpu.async_copy(
                rows_v.at[b, pl.ds(0, sz)],
                out_hbm.at[pl.ds(base + off, sz)], ssems[b])

        for j in range(GDEPTH):
            fire_gather(j)
        for j in range(n):
            if j + GDEPTH < n:
                fire_gather(j + GDEPTH)
            fire_scatter(j)
        for j in range(n - NBUF, n):
            scat[j].wait()

    return k(ids, table)


def kernel(entity_ids, table):
    ids = jnp.squeeze(entity_ids).astype(jnp.int32)
    return _sc_gather(ids, table)
